# baseline (speedup 1.0000x reference)
import jax
import jax.numpy as jnp
from jax.experimental import pallas as pl
from jax.experimental.pallas import tpu as pltpu


def _pos_kernel(row_ref, col_ref, out_ref, s0, s1, s2, s3, sem):
    h = row_ref.shape[0]
    w = col_ref.shape[0]
    f_half = row_ref.shape[1]
    col_t = jnp.transpose(col_ref[...], (1, 0))
    row_t = jnp.transpose(row_ref[...], (1, 0))
    scratches = [s0, s1, s2, s3]
    for scratch in scratches:
        for i in range(h):
            scratch[0:f_half, i * w:(i + 1) * w] = col_t
            scratch[f_half:2 * f_half, i * w:(i + 1) * w] = jnp.broadcast_to(
                row_t[:, i:i + 1], (f_half, w)
            )
    b = out_ref.shape[0]
    copies = [
        pltpu.make_async_copy(scratches[i % 4], out_ref.at[i], sem.at[i])
        for i in range(b)
    ]
    for c in copies:
        c.start()
    for c in copies:
        c.wait()


def kernel(mask, row_embed, col_embed):
    b, h, w = mask.shape
    f_half = row_embed.shape[1]
    f = 2 * f_half
    out = pl.pallas_call(
        _pos_kernel,
        out_specs=pl.BlockSpec(memory_space=pl.ANY),
        out_shape=jax.ShapeDtypeStruct((b, f, h * w), jnp.float32),
        scratch_shapes=[
            pltpu.VMEM((f, h * w), jnp.float32),
            pltpu.VMEM((f, h * w), jnp.float32),
            pltpu.VMEM((f, h * w), jnp.float32),
            pltpu.VMEM((f, h * w), jnp.float32),
            pltpu.SemaphoreType.DMA((32,)),
        ],
    )(row_embed, col_embed)
    return out.reshape(b, f, h, w)
